# Initial kernel scaffold; baseline (speedup 1.0000x reference)
import functools

import jax
import jax.numpy as jnp
from jax import lax
from jax.experimental import pallas as pl
from jax.experimental.pallas import tpu as pltpu
from jax.experimental.pallas import tpu_sc as plsc

VOCAB = 1000000
EMBED_DIM = 32
BATCH = 16384
HIST = 50
NUM_IDS = BATCH * HIST

NC, NS = 2, 16
NW = NC * NS
IDS_PER_W = NUM_IDS // NW
IROWS = 128
KCH = 20
CH = KCH * IROWS
NCH = IDS_PER_W // CH
ROWS_PER_W = IDS_PER_W // IROWS

_mesh = plsc.VectorSubcoreMesh(core_axis_name="c", subcore_axis_name="s")


@functools.partial(
    pl.kernel,
    mesh=_mesh,
    out_type=jax.ShapeDtypeStruct((NUM_IDS, EMBED_DIM), jnp.float32),
    scratch_types=[
        pltpu.VMEM((KCH, IROWS), jnp.int32),
        pltpu.VMEM((CH, EMBED_DIM), jnp.float32),
        pltpu.SemaphoreType.DMA,
    ],
)
def _gather_sc(idx_hbm, table_hbm, out_hbm, idx_v, rows_v, sem):
    wid = lax.axis_index("s") * NC + lax.axis_index("c")
    row_base = wid * ROWS_PER_W
    out_base = wid * IDS_PER_W

    def chunk(g, carry):
        pltpu.sync_copy(idx_hbm.at[pl.ds(row_base + g * KCH, KCH)], idx_v)
        copies = []
        for j in range(KCH):
            copies.append(
                pltpu.async_copy(
                    table_hbm.at[idx_v.at[j]],
                    rows_v.at[pl.ds(j * IROWS, IROWS)],
                    sem,
                )
            )
        for c in copies:
            c.wait()
        pltpu.sync_copy(rows_v, out_hbm.at[pl.ds(out_base + g * CH, CH)])
        return carry

    lax.fori_loop(0, NCH, chunk, 0)


def kernel(input_ids, weight):
    idx = jnp.asarray(input_ids, jnp.int32).reshape(NUM_IDS // IROWS, IROWS)
    out = _gather_sc(idx, weight)
    return out.reshape(BATCH, HIST, EMBED_DIM)


# speedup vs baseline: 1.0943x; 1.0943x over previous
import functools

import jax
import jax.numpy as jnp
from jax import lax
from jax.experimental import pallas as pl
from jax.experimental.pallas import tpu as pltpu
from jax.experimental.pallas import tpu_sc as plsc

VOCAB = 1000000
EMBED_DIM = 32
BATCH = 16384
HIST = 50
NUM_IDS = BATCH * HIST

NC, NS = 2, 16
NW = NC * NS
IDS_PER_W = NUM_IDS // NW
IROWS = 128
KCH = 8
CH = KCH * IROWS
NCH = IDS_PER_W // CH
ROWS_PER_W = IDS_PER_W // IROWS

_mesh = plsc.VectorSubcoreMesh(core_axis_name="c", subcore_axis_name="s")


@functools.partial(
    pl.kernel,
    mesh=_mesh,
    out_type=jax.ShapeDtypeStruct((NUM_IDS, EMBED_DIM), jnp.float32),
    scratch_types=[
        pltpu.VMEM((KCH, IROWS), jnp.int32),
        pltpu.VMEM((CH, EMBED_DIM), jnp.float32),
        pltpu.SemaphoreType.DMA,
    ],
    compiler_params=pltpu.CompilerParams(use_tc_tiling_on_sc=False),
)
def _gather_sc(idx_hbm, table_hbm, out_hbm, idx_v, rows_v, sem):
    wid = lax.axis_index("s") * NC + lax.axis_index("c")
    row_base = wid * ROWS_PER_W
    out_base = wid * IDS_PER_W

    def chunk(g, carry):
        pltpu.sync_copy(idx_hbm.at[pl.ds(row_base + g * KCH, KCH)], idx_v)
        copies = []
        for j in range(KCH):
            copies.append(
                pltpu.async_copy(
                    table_hbm.at[idx_v.at[j]],
                    rows_v.at[pl.ds(j * IROWS, IROWS)],
                    sem,
                )
            )
        for c in copies:
            c.wait()
        pltpu.sync_copy(rows_v, out_hbm.at[pl.ds(out_base + g * CH, CH)])
        return carry

    lax.fori_loop(0, NCH, chunk, 0)


def kernel(input_ids, weight):
    idx = jnp.asarray(input_ids, jnp.int32).reshape(NUM_IDS // IROWS, IROWS)
    out = _gather_sc(idx, weight)
    return out.reshape(BATCH, HIST, EMBED_DIM)


# speedup vs baseline: 1.1035x; 1.0084x over previous
import functools

import jax
import jax.numpy as jnp
from jax import lax
from jax.experimental import pallas as pl
from jax.experimental.pallas import tpu as pltpu
from jax.experimental.pallas import tpu_sc as plsc

VOCAB = 1000000
EMBED_DIM = 32
BATCH = 16384
HIST = 50
NUM_IDS = BATCH * HIST

NC, NS = 2, 16
NW = NC * NS
IDS_PER_W = NUM_IDS // NW
CH = 1024
NCH = IDS_PER_W // CH

_mesh = plsc.VectorSubcoreMesh(core_axis_name="c", subcore_axis_name="s")


@functools.partial(
    pl.kernel,
    mesh=_mesh,
    out_type=jax.ShapeDtypeStruct((NUM_IDS, EMBED_DIM), jnp.float32),
    scratch_types=[
        pltpu.VMEM((IDS_PER_W,), jnp.int32),
        pltpu.VMEM((CH, EMBED_DIM), jnp.float32),
        pltpu.SemaphoreType.DMA,
    ],
    compiler_params=pltpu.CompilerParams(use_tc_tiling_on_sc=False),
)
def _gather_sc(idx_hbm, table_hbm, out_hbm, idx_v, rows_v, sem):
    wid = lax.axis_index("s") * NC + lax.axis_index("c")
    base = wid * IDS_PER_W

    pltpu.sync_copy(idx_hbm.at[pl.ds(base, IDS_PER_W)], idx_v)

    def chunk(g, carry):
        pltpu.async_copy(
            table_hbm.at[idx_v.at[pl.ds(g * CH, CH)]],
            rows_v,
            sem,
        ).wait()
        pltpu.sync_copy(rows_v, out_hbm.at[pl.ds(base + g * CH, CH)])
        return carry

    lax.fori_loop(0, NCH, chunk, 0)


def kernel(input_ids, weight):
    idx = jnp.asarray(input_ids, jnp.int32).reshape(NUM_IDS)
    out = _gather_sc(idx, weight)
    return out.reshape(BATCH, HIST, EMBED_DIM)


# speedup vs baseline: 1.1095x; 1.0054x over previous
import functools

import jax
import jax.numpy as jnp
from jax import lax
from jax.experimental import pallas as pl
from jax.experimental.pallas import tpu as pltpu
from jax.experimental.pallas import tpu_sc as plsc

VOCAB = 1000000
EMBED_DIM = 32
BATCH = 16384
HIST = 50
NUM_IDS = BATCH * HIST

NC, NS = 2, 16
NW = NC * NS
IDS_PER_W = NUM_IDS // NW
SUB = 128
KCH = 10
CH = KCH * SUB
NCH = IDS_PER_W // CH
NPAIR = NCH // 2

_mesh = plsc.VectorSubcoreMesh(core_axis_name="c", subcore_axis_name="s")


@functools.partial(
    pl.kernel,
    mesh=_mesh,
    out_type=jax.ShapeDtypeStruct((NUM_IDS, EMBED_DIM), jnp.float32),
    scratch_types=[
        pltpu.VMEM((IDS_PER_W,), jnp.int32),
        pltpu.VMEM((CH, EMBED_DIM), jnp.float32),
        pltpu.VMEM((CH, EMBED_DIM), jnp.float32),
        pltpu.SemaphoreType.DMA,
        pltpu.SemaphoreType.DMA,
        pltpu.SemaphoreType.DMA,
        pltpu.SemaphoreType.DMA,
    ],
    compiler_params=pltpu.CompilerParams(use_tc_tiling_on_sc=False),
)
def _gather_sc(idx_hbm, table_hbm, out_hbm, idx_v, rows0_v, rows1_v,
               sg0, sg1, sw0, sw1):
    wid = lax.axis_index("s") * NC + lax.axis_index("c")
    base = wid * IDS_PER_W

    pltpu.sync_copy(idx_hbm.at[pl.ds(base, IDS_PER_W)], idx_v)

    def fire(rows_v, sem, cbase):
        for j in range(KCH):
            pltpu.async_copy(
                table_hbm.at[idx_v.at[pl.ds(cbase + j * SUB, SUB)]],
                rows_v.at[pl.ds(j * SUB, SUB)],
                sem,
            )

    def drain_gather(rows_v, sem):
        pltpu.make_async_copy(table_hbm.at[pl.ds(0, CH)], rows_v, sem).wait()

    def drain_wb(rows_v, sem):
        pltpu.make_async_copy(rows_v, out_hbm.at[pl.ds(base, CH)], sem).wait()

    fire(rows0_v, sg0, 0)
    fire(rows1_v, sg1, CH)

    def pair(i, carry):
        c0 = (2 * i) * CH
        c1 = c0 + CH
        drain_gather(rows0_v, sg0)
        pltpu.async_copy(rows0_v, out_hbm.at[pl.ds(base + c0, CH)], sw0)
        drain_gather(rows1_v, sg1)
        pltpu.async_copy(rows1_v, out_hbm.at[pl.ds(base + c1, CH)], sw1)
        drain_wb(rows0_v, sw0)
        fire(rows0_v, sg0, c1 + CH)
        drain_wb(rows1_v, sw1)
        fire(rows1_v, sg1, c1 + 2 * CH)
        return carry

    lax.fori_loop(0, NPAIR - 1, pair, 0)

    c0 = (NCH - 2) * CH
    drain_gather(rows0_v, sg0)
    pltpu.async_copy(rows0_v, out_hbm.at[pl.ds(base + c0, CH)], sw0)
    drain_gather(rows1_v, sg1)
    pltpu.async_copy(rows1_v, out_hbm.at[pl.ds(base + c0 + CH, CH)], sw1)
    drain_wb(rows0_v, sw0)
    drain_wb(rows1_v, sw1)


def kernel(input_ids, weight):
    idx = jnp.asarray(input_ids, jnp.int32).reshape(NUM_IDS)
    out = _gather_sc(idx, weight)
    return out.reshape(BATCH, HIST, EMBED_DIM)


# speedup vs baseline: 1.1107x; 1.0011x over previous
import functools

import jax
import jax.numpy as jnp
from jax import lax
from jax.experimental import pallas as pl
from jax.experimental.pallas import tpu as pltpu
from jax.experimental.pallas import tpu_sc as plsc

VOCAB = 1000000
EMBED_DIM = 32
BATCH = 16384
HIST = 50
NUM_IDS = BATCH * HIST

NC, NS = 2, 16
NW = NC * NS
SUB = 128
KCH = 10
CH = KCH * SUB

_mesh = plsc.VectorSubcoreMesh(core_axis_name="c", subcore_axis_name="s")


def _make_gather(n_ids):
    ids_per_w = n_ids // NW
    nch = ids_per_w // CH
    npair = nch // 2

    @functools.partial(
        pl.kernel,
        mesh=_mesh,
        out_type=jax.ShapeDtypeStruct((n_ids, EMBED_DIM), jnp.float32),
        scratch_types=[
            pltpu.VMEM((ids_per_w,), jnp.int32),
            pltpu.VMEM((CH, EMBED_DIM), jnp.float32),
            pltpu.VMEM((CH, EMBED_DIM), jnp.float32),
            pltpu.SemaphoreType.DMA,
            pltpu.SemaphoreType.DMA,
            pltpu.SemaphoreType.DMA,
            pltpu.SemaphoreType.DMA,
        ],
        compiler_params=pltpu.CompilerParams(use_tc_tiling_on_sc=False),
    )
    def _gather_sc(idx_hbm, table_hbm, out_hbm, idx_v, rows0_v, rows1_v,
                   sg0, sg1, sw0, sw1):
        wid = lax.axis_index("s") * NC + lax.axis_index("c")
        base = wid * ids_per_w

        pltpu.sync_copy(idx_hbm.at[pl.ds(base, ids_per_w)], idx_v)

        def fire(rows_v, sem, cbase):
            for j in range(KCH):
                pltpu.async_copy(
                    table_hbm.at[idx_v.at[pl.ds(cbase + j * SUB, SUB)]],
                    rows_v.at[pl.ds(j * SUB, SUB)],
                    sem,
                )

        def drain_gather(rows_v, sem):
            pltpu.make_async_copy(table_hbm.at[pl.ds(0, CH)], rows_v,
                                  sem).wait()

        def drain_wb(rows_v, sem):
            pltpu.make_async_copy(rows_v, out_hbm.at[pl.ds(base, CH)],
                                  sem).wait()

        fire(rows0_v, sg0, 0)
        fire(rows1_v, sg1, CH)

        def pair(i, carry):
            c0 = (2 * i) * CH
            c1 = c0 + CH
            drain_gather(rows0_v, sg0)
            pltpu.async_copy(rows0_v, out_hbm.at[pl.ds(base + c0, CH)], sw0)
            drain_gather(rows1_v, sg1)
            pltpu.async_copy(rows1_v, out_hbm.at[pl.ds(base + c1, CH)], sw1)
            drain_wb(rows0_v, sw0)
            fire(rows0_v, sg0, c1 + CH)
            drain_wb(rows1_v, sw1)
            fire(rows1_v, sg1, c1 + 2 * CH)
            return carry

        lax.fori_loop(0, npair - 1, pair, 0)

        c0 = (nch - 2) * CH
        drain_gather(rows0_v, sg0)
        pltpu.async_copy(rows0_v, out_hbm.at[pl.ds(base + c0, CH)], sw0)
        drain_gather(rows1_v, sg1)
        pltpu.async_copy(rows1_v, out_hbm.at[pl.ds(base + c0 + CH, CH)], sw1)
        drain_wb(rows0_v, sw0)
        drain_wb(rows1_v, sw1)

    return _gather_sc


_gather_half = _make_gather(NUM_IDS // 2)


def kernel(input_ids, weight):
    idx = jnp.asarray(input_ids, jnp.int32).reshape(NUM_IDS)
    half = NUM_IDS // 2
    o1 = _gather_half(idx[:half], weight)
    o2 = _gather_half(idx[half:], weight)
    out = jnp.concatenate([o1, o2], axis=0)
    return out.reshape(BATCH, HIST, EMBED_DIM)
